# Initial kernel scaffold; baseline (speedup 1.0000x reference)
import functools

import jax
import jax.numpy as jnp
from jax import lax
from jax.experimental import pallas as pl
from jax.experimental.pallas import tpu as pltpu
from jax.experimental.pallas import tpu_sc as plsc

F32 = jnp.float32
_NC = 2
_NS = 16
_NW = _NC * _NS
_H = 16
_CHUNK = 2000
_BR = 1024


def _sc_segsum(np_, ep, values, src, dst, zrows, ones_rows):
    epw = ep // _NW
    nchunks = epw // _CHUNK
    zr = np_ // _NS
    npw = np_ // _NW
    mesh = plsc.VectorSubcoreMesh(core_axis_name="c", subcore_axis_name="s")
    with_gather = values is not None

    def body(*refs):
        if with_gather:
            (hs_h, src_h, dst_h, z_h, out_h,
             sidx, didx, rows, obuf, agg, sem) = refs
        else:
            (dst_h, o_h, z_h, out_h,
             didx, rows, obuf, agg, sem) = refs
        c = lax.axis_index("c")
        s = lax.axis_index("s")
        w = s * _NC + c
        pltpu.sync_copy(z_h, rows.at[pl.ds(0, zr)])
        pltpu.sync_copy(rows.at[pl.ds(0, zr)], agg.at[pl.ds(s * zr, zr)])
        plsc.subcore_barrier()
        if not with_gather:
            pltpu.sync_copy(o_h, rows)
        ebase = w * epw
        for k in range(nchunks):
            off = ebase + k * _CHUNK
            pltpu.sync_copy(dst_h.at[pl.ds(off, _CHUNK)], didx)
            if with_gather:
                pltpu.sync_copy(src_h.at[pl.ds(off, _CHUNK)], sidx)
                pltpu.async_copy(hs_h.at[sidx], rows, sem).wait()
            pltpu.sync_copy(rows, agg.at[didx], add=True)
        plsc.subcore_barrier()
        nbase = w * npw
        pltpu.sync_copy(agg.at[pl.ds(nbase, npw)], obuf)
        pltpu.sync_copy(obuf, out_h.at[pl.ds(c * np_ + nbase, npw)])

    scratch = []
    if with_gather:
        scratch.append(pltpu.VMEM((_CHUNK,), jnp.int32))
    scratch += [
        pltpu.VMEM((_CHUNK,), jnp.int32),
        pltpu.VMEM((_CHUNK, _H), F32),
        pltpu.VMEM((npw, _H), F32),
        pltpu.VMEM_SHARED((np_, _H), F32),
        pltpu.SemaphoreType.DMA,
    ]
    fn = functools.partial(
        pl.kernel,
        out_type=jax.ShapeDtypeStruct((2 * np_, _H), F32),
        mesh=mesh,
        scratch_types=scratch,
    )(body)
    if with_gather:
        return fn(values, src, dst, zrows)
    return fn(dst, ones_rows, zrows)


def _tc_call(body, grid, in_arrays, in_specs, out_shapes, out_specs):
    return pl.pallas_call(
        body,
        grid=grid,
        in_specs=in_specs,
        out_specs=out_specs,
        out_shape=out_shapes,
    )(*in_arrays)


def kernel(features, edge_index, W0, b0, W1, b1):
    n, f = features.shape
    e = edge_index.shape[1]
    h = W0.shape[2]
    ncls = W1.shape[2]
    assert h == _H
    np_ = ((n + _BR - 1) // _BR) * _BR
    assert np_ % (_NW * 8) == 0 and np_ % _NS == 0
    ep = ((e + _NW * _CHUNK - 1) // (_NW * _CHUNK)) * (_NW * _CHUNK)
    if ep > e:
        assert np_ > n
    nb = np_ // _BR

    src = edge_index[0]
    dst = edge_index[1]
    if ep > e:
        src = jnp.pad(src, (0, ep - e), constant_values=n)
        dst = jnp.pad(dst, (0, ep - e), constant_values=np_ - 1)
    xp = jnp.pad(features, ((0, np_ - n), (0, 0)))
    wcat = jnp.concatenate([W0[0] - W0[2], W0[1], W0[2]], axis=1)
    w1a = W1[0] - W1[2]
    w1b = W1[1]
    w1c = 2.0 * W1[2]
    b0r = b0.reshape(1, h)
    b1r = b1.reshape(1, ncls)
    zrows = jnp.zeros((np_ // _NS, _H), F32)
    ones_rows = jnp.ones((_CHUNK, _H), F32)

    row = lambda: pl.BlockSpec((_BR, _H), lambda i: (i, 0))
    rowlo = row
    rowhi = lambda: pl.BlockSpec((_BR, _H), lambda i: (i + nb, 0))

    degp = _sc_segsum(np_, ep, None, src, dst, zrows, ones_rows)

    def m0(x_r, w_r, d0_r, d1_r, y_r, cs_r, nt_r):
        norm = lax.rsqrt(jnp.maximum(d0_r[...] + d1_r[...], 1.0))
        y = jnp.dot(x_r[...], w_r[...], preferred_element_type=F32)
        y_r[...] = y
        cs_r[...] = y[:, 2 * h:3 * h] * norm
        nt_r[...] = norm

    y, cs, normt = _tc_call(
        m0, (nb,),
        [xp, wcat, degp, degp],
        [pl.BlockSpec((_BR, f), lambda i: (i, 0)),
         pl.BlockSpec((f, 3 * h), lambda i: (0, 0)),
         rowlo(), rowhi()],
        [jax.ShapeDtypeStruct((np_, 3 * h), F32),
         jax.ShapeDtypeStruct((np_, h), F32),
         jax.ShapeDtypeStruct((np_, h), F32)],
        [pl.BlockSpec((_BR, 3 * h), lambda i: (i, 0)), row(), row()],
    )

    agg1 = _sc_segsum(np_, ep, cs, src, dst, zrows, ones_rows)

    def c1(g0_r, g1_r, y_r, nt_r, ss_r):
        norm = nt_r[...]
        g = g0_r[...] + g1_r[...]
        a = y_r[...][:, h:2 * h]
        ss_r[...] = norm * a - 2.0 * (norm * norm) * g

    (ss,) = _tc_call(
        c1, (nb,),
        [agg1, agg1, y, normt],
        [rowlo(), rowhi(),
         pl.BlockSpec((_BR, 3 * h), lambda i: (i, 0)), row()],
        [jax.ShapeDtypeStruct((np_, h), F32)],
        [row()],
    )

    agg2 = _sc_segsum(np_, ep, ss, src, dst, zrows, ones_rows)

    def c2(g0_r, g1_r, y_r, nt_r, b0_r, h_r, hs_r):
        norm = nt_r[...]
        q = -norm * (g0_r[...] + g1_r[...])
        hh = jnp.maximum(y_r[...][:, 0:h] + q + b0_r[...], 0.0)
        h_r[...] = hh
        hs_r[...] = norm * hh

    h0, hs = _tc_call(
        c2, (nb,),
        [agg2, agg2, y, normt, b0r],
        [rowlo(), rowhi(),
         pl.BlockSpec((_BR, 3 * h), lambda i: (i, 0)), row(),
         pl.BlockSpec((1, h), lambda i: (0, 0))],
        [jax.ShapeDtypeStruct((np_, h), F32),
         jax.ShapeDtypeStruct((np_, h), F32)],
        [row(), row()],
    )

    agg3 = _sc_segsum(np_, ep, hs, src, dst, zrows, ones_rows)

    def c3(g0_r, g1_r, nt_r, t1_r, t1s_r):
        norm = nt_r[...]
        t1 = -norm * (g0_r[...] + g1_r[...])
        t1_r[...] = t1
        t1s_r[...] = norm * t1

    t1, t1s = _tc_call(
        c3, (nb,),
        [agg3, agg3, normt],
        [rowlo(), rowhi(), row()],
        [jax.ShapeDtypeStruct((np_, h), F32),
         jax.ShapeDtypeStruct((np_, h), F32)],
        [row(), row()],
    )

    agg4 = _sc_segsum(np_, ep, t1s, src, dst, zrows, ones_rows)

    def m1(g0_r, g1_r, nt_r, h_r, t1_r, wa_r, wb_r, wc_r, b1_r, o_r):
        pt1 = -nt_r[...] * (g0_r[...] + g1_r[...])
        acc = jnp.dot(h_r[...], wa_r[...], preferred_element_type=F32)
        acc += jnp.dot(t1_r[...], wb_r[...], preferred_element_type=F32)
        acc += jnp.dot(pt1, wc_r[...], preferred_element_type=F32)
        o_r[...] = jnp.maximum(acc + b1_r[...], 0.0)

    (out,) = _tc_call(
        m1, (nb,),
        [agg4, agg4, normt, h0, t1, w1a, w1b, w1c, b1r],
        [rowlo(), rowhi(), row(), row(), row(),
         pl.BlockSpec((h, ncls), lambda i: (0, 0)),
         pl.BlockSpec((h, ncls), lambda i: (0, 0)),
         pl.BlockSpec((h, ncls), lambda i: (0, 0)),
         pl.BlockSpec((1, ncls), lambda i: (0, 0))],
        [jax.ShapeDtypeStruct((np_, ncls), F32)],
        [pl.BlockSpec((_BR, ncls), lambda i: (i, 0))],
    )
    return out[:n]


# speedup vs baseline: 25.1692x; 25.1692x over previous
import functools

import jax
import jax.numpy as jnp
from jax import lax
from jax.experimental import pallas as pl
from jax.experimental.pallas import tpu as pltpu
from jax.experimental.pallas import tpu_sc as plsc

F32 = jnp.float32
_NC = 2
_NS = 16
_NW = _NC * _NS
_H = 16
_CHUNK = 2000
_BR = 1024


def _sc_segsum(np_, ep, values, src, dst, zrows, ones_rows):
    epw = ep // _NW
    nchunks = epw // _CHUNK
    zr = np_ // _NS
    npw = np_ // _NW
    mesh = plsc.VectorSubcoreMesh(core_axis_name="c", subcore_axis_name="s")
    with_gather = values is not None

    def body(*refs):
        if with_gather:
            (hs_h, src_h, dst_h, z_h, out_h,
             sidx, didx, rows, obuf, agg, sem) = refs
        else:
            (dst_h, o_h, z_h, out_h,
             didx, rows, obuf, agg, sem) = refs
        c = lax.axis_index("c")
        s = lax.axis_index("s")
        w = s * _NC + c
        pltpu.sync_copy(z_h, rows.at[pl.ds(0, zr)])
        pltpu.sync_copy(rows.at[pl.ds(0, zr)], agg.at[pl.ds(s * zr, zr)])
        plsc.subcore_barrier()
        if not with_gather:
            pltpu.sync_copy(o_h, rows)
        ebase = w * epw
        for k in range(nchunks):
            off = ebase + k * _CHUNK
            pltpu.sync_copy(dst_h.at[pl.ds(off, _CHUNK)], didx)
            if with_gather:
                pltpu.sync_copy(src_h.at[pl.ds(off, _CHUNK)], sidx)
                pltpu.async_copy(hs_h.at[sidx], rows, sem).wait()
            pltpu.sync_copy(rows, agg.at[didx], add=True)
        plsc.subcore_barrier()
        nbase = s * zr
        pltpu.sync_copy(agg.at[pl.ds(nbase, zr)], obuf)
        pltpu.sync_copy(obuf, out_h.at[pl.ds(c * np_ + nbase, zr)])

    scratch = []
    if with_gather:
        scratch.append(pltpu.VMEM((_CHUNK,), jnp.int32))
    scratch += [
        pltpu.VMEM((_CHUNK,), jnp.int32),
        pltpu.VMEM((_CHUNK, _H), F32),
        pltpu.VMEM((zr, _H), F32),
        pltpu.VMEM_SHARED((np_, _H), F32),
        pltpu.SemaphoreType.DMA,
    ]
    fn = functools.partial(
        pl.kernel,
        out_type=jax.ShapeDtypeStruct((2 * np_, _H), F32),
        mesh=mesh,
        scratch_types=scratch,
        compiler_params=pltpu.CompilerParams(use_tc_tiling_on_sc=False),
    )(body)
    if with_gather:
        return fn(values, src, dst, zrows)
    return fn(dst, ones_rows, zrows)


def _tc_call(body, grid, in_arrays, in_specs, out_shapes, out_specs):
    return pl.pallas_call(
        body,
        grid=grid,
        in_specs=in_specs,
        out_specs=out_specs,
        out_shape=out_shapes,
    )(*in_arrays)


def kernel(features, edge_index, W0, b0, W1, b1):
    n, f = features.shape
    e = edge_index.shape[1]
    h = W0.shape[2]
    ncls = W1.shape[2]
    assert h == _H
    np_ = ((n + _BR - 1) // _BR) * _BR
    assert np_ % (_NW * 8) == 0 and np_ % _NS == 0
    ep = ((e + _NW * _CHUNK - 1) // (_NW * _CHUNK)) * (_NW * _CHUNK)
    if ep > e:
        assert np_ > n
    nb = np_ // _BR

    src = edge_index[0]
    dst = edge_index[1]
    if ep > e:
        src = jnp.pad(src, (0, ep - e), constant_values=n)
        dst = jnp.pad(dst, (0, ep - e), constant_values=np_ - 1)
    xp = jnp.pad(features, ((0, np_ - n), (0, 0)))
    wcat = jnp.concatenate([W0[0] - W0[2], W0[1], W0[2]], axis=1)
    w1a = W1[0] - W1[2]
    w1b = W1[1]
    w1c = 2.0 * W1[2]
    b0r = b0.reshape(1, h)
    b1r = b1.reshape(1, ncls)
    zrows = jnp.zeros((np_ // _NS, _H), F32)
    ones_rows = jnp.ones((_CHUNK, _H), F32)

    row = lambda: pl.BlockSpec((_BR, _H), lambda i: (i, 0))
    rowlo = row
    rowhi = lambda: pl.BlockSpec((_BR, _H), lambda i: (i + nb, 0))

    degp = _sc_segsum(np_, ep, None, src, dst, zrows, ones_rows)

    def m0(x_r, w_r, d0_r, d1_r, y_r, cs_r, nt_r):
        norm = lax.rsqrt(jnp.maximum(d0_r[...] + d1_r[...], 1.0))
        y = jnp.dot(x_r[...], w_r[...], preferred_element_type=F32)
        y_r[...] = y
        cs_r[...] = y[:, 2 * h:3 * h] * norm
        nt_r[...] = norm

    y, cs, normt = _tc_call(
        m0, (nb,),
        [xp, wcat, degp, degp],
        [pl.BlockSpec((_BR, f), lambda i: (i, 0)),
         pl.BlockSpec((f, 3 * h), lambda i: (0, 0)),
         rowlo(), rowhi()],
        [jax.ShapeDtypeStruct((np_, 3 * h), F32),
         jax.ShapeDtypeStruct((np_, h), F32),
         jax.ShapeDtypeStruct((np_, h), F32)],
        [pl.BlockSpec((_BR, 3 * h), lambda i: (i, 0)), row(), row()],
    )

    agg1 = _sc_segsum(np_, ep, cs, src, dst, zrows, ones_rows)

    def c1(g0_r, g1_r, y_r, nt_r, ss_r):
        norm = nt_r[...]
        g = g0_r[...] + g1_r[...]
        a = y_r[...][:, h:2 * h]
        ss_r[...] = norm * a - 2.0 * (norm * norm) * g

    (ss,) = _tc_call(
        c1, (nb,),
        [agg1, agg1, y, normt],
        [rowlo(), rowhi(),
         pl.BlockSpec((_BR, 3 * h), lambda i: (i, 0)), row()],
        [jax.ShapeDtypeStruct((np_, h), F32)],
        [row()],
    )

    agg2 = _sc_segsum(np_, ep, ss, src, dst, zrows, ones_rows)

    def c2(g0_r, g1_r, y_r, nt_r, b0_r, h_r, hs_r):
        norm = nt_r[...]
        q = -norm * (g0_r[...] + g1_r[...])
        hh = jnp.maximum(y_r[...][:, 0:h] + q + b0_r[...], 0.0)
        h_r[...] = hh
        hs_r[...] = norm * hh

    h0, hs = _tc_call(
        c2, (nb,),
        [agg2, agg2, y, normt, b0r],
        [rowlo(), rowhi(),
         pl.BlockSpec((_BR, 3 * h), lambda i: (i, 0)), row(),
         pl.BlockSpec((1, h), lambda i: (0, 0))],
        [jax.ShapeDtypeStruct((np_, h), F32),
         jax.ShapeDtypeStruct((np_, h), F32)],
        [row(), row()],
    )

    agg3 = _sc_segsum(np_, ep, hs, src, dst, zrows, ones_rows)

    def c3(g0_r, g1_r, nt_r, t1_r, t1s_r):
        norm = nt_r[...]
        t1 = -norm * (g0_r[...] + g1_r[...])
        t1_r[...] = t1
        t1s_r[...] = norm * t1

    t1, t1s = _tc_call(
        c3, (nb,),
        [agg3, agg3, normt],
        [rowlo(), rowhi(), row()],
        [jax.ShapeDtypeStruct((np_, h), F32),
         jax.ShapeDtypeStruct((np_, h), F32)],
        [row(), row()],
    )

    agg4 = _sc_segsum(np_, ep, t1s, src, dst, zrows, ones_rows)

    def m1(g0_r, g1_r, nt_r, h_r, t1_r, wa_r, wb_r, wc_r, b1_r, o_r):
        pt1 = -nt_r[...] * (g0_r[...] + g1_r[...])
        acc = jnp.dot(h_r[...], wa_r[...], preferred_element_type=F32)
        acc += jnp.dot(t1_r[...], wb_r[...], preferred_element_type=F32)
        acc += jnp.dot(pt1, wc_r[...], preferred_element_type=F32)
        o_r[...] = jnp.maximum(acc + b1_r[...], 0.0)

    (out,) = _tc_call(
        m1, (nb,),
        [agg4, agg4, normt, h0, t1, w1a, w1b, w1c, b1r],
        [rowlo(), rowhi(), row(), row(), row(),
         pl.BlockSpec((h, ncls), lambda i: (0, 0)),
         pl.BlockSpec((h, ncls), lambda i: (0, 0)),
         pl.BlockSpec((h, ncls), lambda i: (0, 0)),
         pl.BlockSpec((1, ncls), lambda i: (0, 0))],
        [jax.ShapeDtypeStruct((np_, ncls), F32)],
        [pl.BlockSpec((_BR, ncls), lambda i: (i, 0))],
    )
    return out[:n]


# speedup vs baseline: 39.6481x; 1.5753x over previous
import functools

import jax
import jax.numpy as jnp
from jax import lax
from jax.experimental import pallas as pl
from jax.experimental.pallas import tpu as pltpu
from jax.experimental.pallas import tpu_sc as plsc

F32 = jnp.float32
_NC = 2
_NS = 16
_NW = _NC * _NS
_H = 16
_CHUNK = 2000
_BR = 1024
_PB = _BR // 8


def _sc_segsum(np_, ep, values, ei, zrows, ones_rows):
    epw = ep // _NW
    nchunks = epw // _CHUNK
    zr = np_ // _NS
    mesh = plsc.VectorSubcoreMesh(core_axis_name="c", subcore_axis_name="s")
    with_gather = values is not None

    def body(*refs):
        if with_gather:
            (hs_h, ei_h, z_h, out_h,
             sidx, didx, rows0, rows1, obuf, agg, sem0, sem1) = refs
        else:
            (ei_h, o_h, z_h, out_h, didx, rows0, obuf, agg) = refs
        c = lax.axis_index("c")
        s = lax.axis_index("s")
        w = s * _NC + c
        pltpu.sync_copy(z_h, rows0.at[pl.ds(0, zr)])
        pltpu.sync_copy(rows0.at[pl.ds(0, zr)], agg.at[pl.ds(s * zr, zr)])
        plsc.subcore_barrier()
        ebase = w * epw
        if with_gather:
            pltpu.sync_copy(ei_h.at[0, pl.ds(ebase, epw)], sidx)
            for k in range(nchunks):
                pltpu.sync_copy(
                    ei_h.at[1, pl.ds(ebase + k * _CHUNK, _CHUNK)], didx.at[k])
            rows = [rows0, rows1]
            sems = [sem0, sem1]
            descs = [None, None]
            for k in range(min(2, nchunks)):
                descs[k] = pltpu.async_copy(
                    hs_h.at[sidx.at[pl.ds(k * _CHUNK, _CHUNK)]],
                    rows[k], sems[k])
            for k in range(nchunks):
                b = k % 2
                descs[b].wait()
                pltpu.sync_copy(rows[b], agg.at[didx.at[k]], add=True)
                if k + 2 < nchunks:
                    descs[b] = pltpu.async_copy(
                        hs_h.at[sidx.at[pl.ds((k + 2) * _CHUNK, _CHUNK)]],
                        rows[b], sems[b])
        else:
            pltpu.sync_copy(o_h, rows0)
            for k in range(nchunks):
                pltpu.sync_copy(
                    ei_h.at[1, pl.ds(ebase + k * _CHUNK, _CHUNK)], didx.at[k])
                pltpu.sync_copy(rows0, agg.at[didx.at[k]], add=True)
        plsc.subcore_barrier()
        nbase = s * zr
        pltpu.sync_copy(agg.at[pl.ds(nbase, zr)], obuf)
        pltpu.sync_copy(obuf, out_h.at[pl.ds(c * np_ + nbase, zr)])

    scratch = []
    if with_gather:
        scratch.append(pltpu.VMEM((epw,), jnp.int32))
    scratch.append(pltpu.VMEM((nchunks, _CHUNK), jnp.int32))
    scratch.append(pltpu.VMEM((_CHUNK, _H), F32))
    if with_gather:
        scratch.append(pltpu.VMEM((_CHUNK, _H), F32))
    scratch += [
        pltpu.VMEM((zr, _H), F32),
        pltpu.VMEM_SHARED((np_, _H), F32),
    ]
    if with_gather:
        scratch += [pltpu.SemaphoreType.DMA, pltpu.SemaphoreType.DMA]
    fn = functools.partial(
        pl.kernel,
        out_type=jax.ShapeDtypeStruct((2 * np_, _H), F32),
        mesh=mesh,
        scratch_types=scratch,
        compiler_params=pltpu.CompilerParams(use_tc_tiling_on_sc=False),
    )(body)
    if with_gather:
        return fn(values, ei, zrows)
    return fn(ei, ones_rows, zrows)


def _tc_call(body, grid, in_arrays, in_specs, out_shapes, out_specs):
    return pl.pallas_call(
        body,
        grid=grid,
        in_specs=in_specs,
        out_specs=out_specs,
        out_shape=out_shapes,
    )(*in_arrays)


def kernel(features, edge_index, W0, b0, W1, b1):
    n, f = features.shape
    e = edge_index.shape[1]
    h = W0.shape[2]
    ncls = W1.shape[2]
    assert h == _H
    np_ = ((n + _BR - 1) // _BR) * _BR
    assert np_ % (_NW * 8) == 0 and np_ % _NS == 0
    ep = ((e + _NW * _CHUNK - 1) // (_NW * _CHUNK)) * (_NW * _CHUNK)
    if ep > e:
        assert np_ > n
    nb = np_ // _BR
    pr = np_ // 8

    ei = edge_index
    if ep > e:
        ei = jnp.concatenate(
            [jnp.pad(ei[0:1], ((0, 0), (0, ep - e)), constant_values=n),
             jnp.pad(ei[1:2], ((0, 0), (0, ep - e)),
                     constant_values=np_ - 1)], axis=0)
    x_p = jnp.pad(features, ((0, np_ - n), (0, 0))).reshape(pr, 8 * f)
    eye8 = jnp.eye(8, dtype=F32)
    t0 = jnp.stack([W0[0] - W0[2], W0[1], W0[2]])
    bd0 = jnp.einsum('ab,tkf->aktbf', eye8, t0).reshape(8 * f, 3 * 8 * h)
    t1w = jnp.stack([W1[0] - W1[2], W1[1], 2.0 * W1[2]])
    bd1 = jnp.einsum('ab,sgf->sagbf', eye8, t1w).reshape(3 * 8 * h, 8 * ncls)
    b0t = jnp.tile(b0.reshape(1, h), (1, 8))
    b1t = jnp.tile(b1.reshape(1, ncls), (1, 8))
    zrows = jnp.zeros((np_ // _NS, _H), F32)
    ones_rows = jnp.ones((_CHUNK, _H), F32)

    pk = lambda: pl.BlockSpec((_PB, 128), lambda i: (i, 0))
    pklo = pk
    pkhi = lambda: pl.BlockSpec((_PB, 128), lambda i: (i + nb, 0))

    degp = _sc_segsum(np_, ep, None, ei, zrows, ones_rows).reshape(2 * pr, 128)

    def m0(x_r, w_r, d0_r, d1_r, dp_r, a_r, cs_r, nt_r):
        norm = lax.rsqrt(jnp.maximum(d0_r[...] + d1_r[...], 1.0))
        yy = jnp.dot(x_r[...], w_r[...], preferred_element_type=F32)
        dp_r[...] = yy[:, 0:128]
        a_r[...] = yy[:, 128:256]
        cs_r[...] = yy[:, 256:384] * norm
        nt_r[...] = norm

    dp, a, cs, normt = _tc_call(
        m0, (nb,),
        [x_p, bd0, degp, degp],
        [pl.BlockSpec((_PB, 8 * f), lambda i: (i, 0)),
         pl.BlockSpec((8 * f, 3 * 128), lambda i: (0, 0)),
         pklo(), pkhi()],
        [jax.ShapeDtypeStruct((pr, 128), F32),
         jax.ShapeDtypeStruct((pr, 128), F32),
         jax.ShapeDtypeStruct((pr, 128), F32),
         jax.ShapeDtypeStruct((pr, 128), F32)],
        [pk(), pk(), pk(), pk()],
    )

    agg1 = _sc_segsum(np_, ep, cs.reshape(np_, h), ei, zrows,
                      ones_rows).reshape(2 * pr, 128)

    def c1(g0_r, g1_r, a_r, nt_r, ss_r):
        norm = nt_r[...]
        g = g0_r[...] + g1_r[...]
        ss_r[...] = norm * a_r[...] - 2.0 * (norm * norm) * g

    (ss,) = _tc_call(
        c1, (nb,),
        [agg1, agg1, a, normt],
        [pklo(), pkhi(), pk(), pk()],
        [jax.ShapeDtypeStruct((pr, 128), F32)],
        [pk()],
    )

    agg2 = _sc_segsum(np_, ep, ss.reshape(np_, h), ei, zrows,
                      ones_rows).reshape(2 * pr, 128)

    def c2(g0_r, g1_r, dp_r, nt_r, b0_r, h_r, hs_r):
        norm = nt_r[...]
        q = -norm * (g0_r[...] + g1_r[...])
        hh = jnp.maximum(dp_r[...] + q + b0_r[...], 0.0)
        h_r[...] = hh
        hs_r[...] = norm * hh

    h0, hs = _tc_call(
        c2, (nb,),
        [agg2, agg2, dp, normt, b0t],
        [pklo(), pkhi(), pk(), pk(),
         pl.BlockSpec((1, 128), lambda i: (0, 0))],
        [jax.ShapeDtypeStruct((pr, 128), F32),
         jax.ShapeDtypeStruct((pr, 128), F32)],
        [pk(), pk()],
    )

    agg3 = _sc_segsum(np_, ep, hs.reshape(np_, h), ei, zrows,
                      ones_rows).reshape(2 * pr, 128)

    def c3(g0_r, g1_r, nt_r, t1_r, t1s_r):
        norm = nt_r[...]
        t1 = -norm * (g0_r[...] + g1_r[...])
        t1_r[...] = t1
        t1s_r[...] = norm * t1

    t1, t1s = _tc_call(
        c3, (nb,),
        [agg3, agg3, normt],
        [pklo(), pkhi(), pk()],
        [jax.ShapeDtypeStruct((pr, 128), F32),
         jax.ShapeDtypeStruct((pr, 128), F32)],
        [pk(), pk()],
    )

    agg4 = _sc_segsum(np_, ep, t1s.reshape(np_, h), ei, zrows,
                      ones_rows).reshape(2 * pr, 128)

    def m1(g0_r, g1_r, nt_r, h_r, t1_r, w_r, b1_r, o_r):
        pt1 = -nt_r[...] * (g0_r[...] + g1_r[...])
        z = jnp.concatenate([h_r[...], t1_r[...], pt1], axis=1)
        acc = jnp.dot(z, w_r[...], preferred_element_type=F32)
        o_r[...] = jnp.maximum(acc + b1_r[...], 0.0)

    (out_p,) = _tc_call(
        m1, (nb,),
        [agg4, agg4, normt, h0, t1, bd1, b1t],
        [pklo(), pkhi(), pk(), pk(), pk(),
         pl.BlockSpec((3 * 8 * h, 8 * ncls), lambda i: (0, 0)),
         pl.BlockSpec((1, 8 * ncls), lambda i: (0, 0))],
        [jax.ShapeDtypeStruct((pr, 8 * ncls), F32)],
        [pl.BlockSpec((_PB, 8 * ncls), lambda i: (i, 0))],
    )
    return out_p.reshape(np_, ncls)[:n]


# speedup vs baseline: 41.2799x; 1.0412x over previous
import functools

import jax
import jax.numpy as jnp
from jax import lax
from jax.experimental import pallas as pl
from jax.experimental.pallas import tpu as pltpu
from jax.experimental.pallas import tpu_sc as plsc

F32 = jnp.float32
_NC = 2
_NS = 16
_NW = _NC * _NS
_H = 16
_CHUNK = 1000
_BR = 1024
_PB = _BR // 8


def _sc_segsum(np_, ep, values, ei, zrows, ones_rows):
    epw = ep // _NW
    nchunks = epw // _CHUNK
    zr = np_ // _NS
    mesh = plsc.VectorSubcoreMesh(core_axis_name="c", subcore_axis_name="s")
    with_gather = values is not None

    def body(*refs):
        if with_gather:
            (hs_h, ei_h, z_h, out_h, sidx, didx, rows0, rows1, rows2,
             obuf, agg, sem0, sem1, sem2, sem3, sem4, sem5) = refs
        else:
            (ei_h, o_h, z_h, out_h, didx, rows0, obuf, agg) = refs
        c = lax.axis_index("c")
        s = lax.axis_index("s")
        w = s * _NC + c
        pltpu.sync_copy(z_h, rows0.at[pl.ds(0, zr)])
        pltpu.sync_copy(rows0.at[pl.ds(0, zr)], agg.at[pl.ds(s * zr, zr)])
        plsc.subcore_barrier()
        ebase = w * epw
        if with_gather:
            pltpu.sync_copy(ei_h.at[0, pl.ds(ebase, epw)], sidx)
            for k in range(nchunks):
                pltpu.sync_copy(
                    ei_h.at[1, pl.ds(ebase + k * _CHUNK, _CHUNK)], didx.at[k])
            rows = [rows0, rows1, rows2]
            gsem = [sem0, sem1, sem2]
            ssem = [sem3, sem4, sem5]
            gd = [None, None, None]
            sd = [None, None, None]
            for k in range(min(2, nchunks)):
                gd[k] = pltpu.async_copy(
                    hs_h.at[sidx.at[pl.ds(k * _CHUNK, _CHUNK)]],
                    rows[k], gsem[k])
            for k in range(nchunks):
                b = k % 3
                gd[b].wait()
                sd[b] = pltpu.async_copy(
                    rows[b], agg.at[didx.at[k]], ssem[b], add=True)
                nk = k + 2
                if nk < nchunks:
                    bn = nk % 3
                    if sd[bn] is not None:
                        sd[bn].wait()
                        sd[bn] = None
                    gd[bn] = pltpu.async_copy(
                        hs_h.at[sidx.at[pl.ds(nk * _CHUNK, _CHUNK)]],
                        rows[bn], gsem[bn])
            for b in range(3):
                if sd[b] is not None:
                    sd[b].wait()
        else:
            pltpu.sync_copy(o_h, rows0)
            for k in range(nchunks):
                pltpu.sync_copy(
                    ei_h.at[1, pl.ds(ebase + k * _CHUNK, _CHUNK)], didx.at[k])
                pltpu.sync_copy(rows0, agg.at[didx.at[k]], add=True)
        plsc.subcore_barrier()
        nbase = s * zr
        pltpu.sync_copy(agg.at[pl.ds(nbase, zr)], obuf)
        pltpu.sync_copy(obuf, out_h.at[pl.ds(c * np_ + nbase, zr)])

    scratch = []
    if with_gather:
        scratch.append(pltpu.VMEM((epw,), jnp.int32))
    scratch.append(pltpu.VMEM((nchunks, _CHUNK), jnp.int32))
    scratch.append(pltpu.VMEM((_CHUNK, _H), F32))
    if with_gather:
        scratch.append(pltpu.VMEM((_CHUNK, _H), F32))
        scratch.append(pltpu.VMEM((_CHUNK, _H), F32))
    scratch += [
        pltpu.VMEM((zr, _H), F32),
        pltpu.VMEM_SHARED((np_, _H), F32),
    ]
    if with_gather:
        scratch += [pltpu.SemaphoreType.DMA] * 6
    fn = functools.partial(
        pl.kernel,
        out_type=jax.ShapeDtypeStruct((2 * np_, _H), F32),
        mesh=mesh,
        scratch_types=scratch,
        compiler_params=pltpu.CompilerParams(use_tc_tiling_on_sc=False),
    )(body)
    if with_gather:
        return fn(values, ei, zrows)
    return fn(ei, ones_rows, zrows)


def _tc_call(body, grid, in_arrays, in_specs, out_shapes, out_specs):
    return pl.pallas_call(
        body,
        grid=grid,
        in_specs=in_specs,
        out_specs=out_specs,
        out_shape=out_shapes,
    )(*in_arrays)


def kernel(features, edge_index, W0, b0, W1, b1):
    n, f = features.shape
    e = edge_index.shape[1]
    h = W0.shape[2]
    ncls = W1.shape[2]
    assert h == _H
    np_ = ((n + _BR - 1) // _BR) * _BR
    assert np_ % (_NW * 8) == 0 and np_ % _NS == 0
    ep = ((e + _NW * _CHUNK - 1) // (_NW * _CHUNK)) * (_NW * _CHUNK)
    if ep > e:
        assert np_ > n
    nb = np_ // _BR
    pr = np_ // 8

    ei = edge_index
    if ep > e:
        ei = jnp.concatenate(
            [jnp.pad(ei[0:1], ((0, 0), (0, ep - e)), constant_values=n),
             jnp.pad(ei[1:2], ((0, 0), (0, ep - e)),
                     constant_values=np_ - 1)], axis=0)
    x_p = jnp.pad(features, ((0, np_ - n), (0, 0))).reshape(pr, 8 * f)
    eye8 = jnp.eye(8, dtype=F32)
    t0 = jnp.stack([W0[0] - W0[2], W0[1], W0[2]])
    bd0 = jnp.einsum('ab,tkf->aktbf', eye8, t0).reshape(8 * f, 3 * 8 * h)
    t1w = jnp.stack([W1[0] - W1[2], W1[1], 2.0 * W1[2]])
    bd1 = jnp.einsum('ab,sgf->sagbf', eye8, t1w).reshape(3 * 8 * h, 8 * ncls)
    b0t = jnp.tile(b0.reshape(1, h), (1, 8))
    b1t = jnp.tile(b1.reshape(1, ncls), (1, 8))
    zrows = jnp.zeros((np_ // _NS, _H), F32)
    ones_rows = jnp.ones((_CHUNK, _H), F32)

    pb = pr // 5
    ng = 5
    pk = lambda: pl.BlockSpec((pb, 128), lambda i: (i, 0))
    pklo = pk
    pkhi = lambda: pl.BlockSpec((pb, 128), lambda i: (i + ng, 0))

    def m0a(x_r, w_r, dp_r, a_r, c_r):
        yy = jnp.dot(x_r[...], w_r[...], preferred_element_type=F32,
                     precision=lax.Precision.HIGHEST)
        dp_r[...] = yy[:, 0:128]
        a_r[...] = yy[:, 128:256]
        c_r[...] = yy[:, 256:384]

    dp, a, cc = _tc_call(
        m0a, (ng,),
        [x_p, bd0],
        [pl.BlockSpec((pb, 8 * f), lambda i: (i, 0)),
         pl.BlockSpec((8 * f, 3 * 128), lambda i: (0, 0))],
        [jax.ShapeDtypeStruct((pr, 128), F32),
         jax.ShapeDtypeStruct((pr, 128), F32),
         jax.ShapeDtypeStruct((pr, 128), F32)],
        [pk(), pk(), pk()],
    )

    degp = _sc_segsum(np_, ep, None, ei, zrows, ones_rows).reshape(2 * pr, 128)

    def m0b(d0_r, d1_r, c_r, cs_r, nt_r):
        norm = 1.0 / jnp.sqrt(jnp.maximum(d0_r[...] + d1_r[...], 1.0))
        cs_r[...] = c_r[...] * norm
        nt_r[...] = norm

    cs, normt = _tc_call(
        m0b, (ng,),
        [degp, degp, cc],
        [pklo(), pkhi(), pk()],
        [jax.ShapeDtypeStruct((pr, 128), F32),
         jax.ShapeDtypeStruct((pr, 128), F32)],
        [pk(), pk()],
    )

    agg1 = _sc_segsum(np_, ep, cs.reshape(np_, h), ei, zrows,
                      ones_rows).reshape(2 * pr, 128)

    def c1(g0_r, g1_r, a_r, nt_r, ss_r):
        norm = nt_r[...]
        g = g0_r[...] + g1_r[...]
        ss_r[...] = norm * a_r[...] - 2.0 * (norm * norm) * g

    (ss,) = _tc_call(
        c1, (ng,),
        [agg1, agg1, a, normt],
        [pklo(), pkhi(), pk(), pk()],
        [jax.ShapeDtypeStruct((pr, 128), F32)],
        [pk()],
    )

    agg2 = _sc_segsum(np_, ep, ss.reshape(np_, h), ei, zrows,
                      ones_rows).reshape(2 * pr, 128)

    def c2(g0_r, g1_r, dp_r, nt_r, b0_r, h_r, hs_r):
        norm = nt_r[...]
        q = -norm * (g0_r[...] + g1_r[...])
        hh = jnp.maximum(dp_r[...] + q + b0_r[...], 0.0)
        h_r[...] = hh
        hs_r[...] = norm * hh

    h0, hs = _tc_call(
        c2, (ng,),
        [agg2, agg2, dp, normt, b0t],
        [pklo(), pkhi(), pk(), pk(),
         pl.BlockSpec((1, 128), lambda i: (0, 0))],
        [jax.ShapeDtypeStruct((pr, 128), F32),
         jax.ShapeDtypeStruct((pr, 128), F32)],
        [pk(), pk()],
    )

    agg3 = _sc_segsum(np_, ep, hs.reshape(np_, h), ei, zrows,
                      ones_rows).reshape(2 * pr, 128)

    def c3(g0_r, g1_r, nt_r, t1_r, t1s_r):
        norm = nt_r[...]
        t1 = -norm * (g0_r[...] + g1_r[...])
        t1_r[...] = t1
        t1s_r[...] = norm * t1

    t1, t1s = _tc_call(
        c3, (ng,),
        [agg3, agg3, normt],
        [pklo(), pkhi(), pk()],
        [jax.ShapeDtypeStruct((pr, 128), F32),
         jax.ShapeDtypeStruct((pr, 128), F32)],
        [pk(), pk()],
    )

    agg4 = _sc_segsum(np_, ep, t1s.reshape(np_, h), ei, zrows,
                      ones_rows).reshape(2 * pr, 128)

    def m1(g0_r, g1_r, nt_r, h_r, t1_r, w_r, b1_r, o_r):
        pt1 = -nt_r[...] * (g0_r[...] + g1_r[...])
        z = jnp.concatenate([h_r[...], t1_r[...], pt1], axis=1)
        acc = jnp.dot(z, w_r[...], preferred_element_type=F32,
                      precision=lax.Precision.HIGHEST)
        o_r[...] = jnp.maximum(acc + b1_r[...], 0.0)

    (out_p,) = _tc_call(
        m1, (ng,),
        [agg4, agg4, normt, h0, t1, bd1, b1t],
        [pklo(), pkhi(), pk(), pk(), pk(),
         pl.BlockSpec((3 * 8 * h, 8 * ncls), lambda i: (0, 0)),
         pl.BlockSpec((1, 8 * ncls), lambda i: (0, 0))],
        [jax.ShapeDtypeStruct((pr, 8 * ncls), F32)],
        [pl.BlockSpec((pb, 8 * ncls), lambda i: (i, 0))],
    )
    return out_p.reshape(np_, ncls)[:n]
